# baseline (speedup 1.0000x reference)
import functools
import math

import jax
import jax.numpy as jnp
import ml_dtypes
import numpy as np
from jax import lax
from jax.experimental import pallas as pl
from jax.experimental.pallas import tpu as pltpu
from jax.experimental.pallas import tpu_sc as plsc

VOCAB = 100000
D = 768
NUM_LANG = 4
B = 4
S = 4096
N = B * S
SCALE = math.sqrt(D)

NSL = 4
S_SL = S // NSL
N_SL = B * S_SL

NC = 2
NS = 16
NW = NC * NS
R_PER_W = N_SL // NW
CHUNK = 32
NBUF = 4
PF = 2
NCHUNK = R_PER_W // CHUNK

BLK = 512
NBLK_SL = S_SL // BLK
NBLK = S // BLK


def _pos_table():
    pos = np.arange(0, S, dtype=np.float32)[:, None]
    div = np.exp(np.arange(0, D, 2, dtype=np.float32) * (-math.log(10000.0) / D))
    pe = np.zeros((S, D), dtype=np.float32)
    pe[:, 0::2] = np.sin(pos * div)
    pe[:, 1::2] = np.cos(pos * div)
    return pe.astype(ml_dtypes.bfloat16)


_PE16 = _pos_table()


def _proj_body(wl_ref, wp_ref, o_ref):
    o_ref[...] = lax.dot_general(
        wl_ref[...], wp_ref[...], (((1,), (1,)), ((), ())),
        preferred_element_type=jnp.float32)


_lang_proj = pl.pallas_call(
    _proj_body,
    out_shape=jax.ShapeDtypeStruct((NUM_LANG, D), jnp.float32),
)


@functools.partial(
    pl.kernel,
    out_type=jax.ShapeDtypeStruct((N_SL, D), jnp.float32),
    mesh=plsc.VectorSubcoreMesh(core_axis_name="c", subcore_axis_name="s"),
    scratch_types=(
        [pltpu.VMEM((R_PER_W,), jnp.int32)]
        + [pltpu.VMEM((CHUNK, D), jnp.float32)] * NBUF
        + [pltpu.SemaphoreType.DMA] * (2 * NBUF)
    ),
)
def _sc_gather(wtok_hbm, tokid_hbm, g_hbm, idx_v, *bufs_and_sems):
    bufs = bufs_and_sems[:NBUF]
    gsems = bufs_and_sems[NBUF:2 * NBUF]
    ssems = bufs_and_sems[2 * NBUF:]
    wid = lax.axis_index("s") * NC + lax.axis_index("c")
    base = wid * R_PER_W
    pltpu.sync_copy(tokid_hbm.at[pl.ds(base, R_PER_W)], idx_v)
    g_cp = [None] * NCHUNK
    st_cp = [None] * NCHUNK
    for c in range(PF):
        g_cp[c] = pltpu.async_copy(
            wtok_hbm.at[idx_v.at[pl.ds(c * CHUNK, CHUNK)]], bufs[c % NBUF],
            gsems[c % NBUF])
    for c in range(NCHUNK):
        k = c % NBUF
        if c + PF < NCHUNK:
            if c >= NBUF - PF:
                st_cp[c - (NBUF - PF)].wait()
            kk = (c + PF) % NBUF
            g_cp[c + PF] = pltpu.async_copy(
                wtok_hbm.at[idx_v.at[pl.ds((c + PF) * CHUNK, CHUNK)]],
                bufs[kk], gsems[kk])
        g_cp[c].wait()
        st_cp[c] = pltpu.async_copy(
            bufs[k], g_hbm.at[pl.ds(base + c * CHUNK, CHUNK)], ssems[k])
    for c in range(max(0, NCHUNK - NBUF), NCHUNK):
        st_cp[c].wait()


def _combine_body(lid_ref, ltab_ref, g_ref, pe_ref, o_ref):
    ids_row = lid_ref[0]
    oh = (lax.broadcasted_iota(jnp.int32, (NUM_LANG, BLK), 0)
          == jnp.broadcast_to(ids_row, (NUM_LANG, BLK))).astype(jnp.float32)
    lang = lax.dot_general(oh, ltab_ref[...], (((0,), (0,)), ((), ())),
                           preferred_element_type=jnp.float32)
    o_ref[...] = (g_ref[...] * SCALE + lang
                  + pe_ref[...].astype(jnp.float32))


def _make_combine(sl, aliased):
    kw = {}
    specs = [
        pl.BlockSpec((1, 1, BLK),
                     lambda i, b: (b * NBLK + sl * NBLK_SL + i, 0, 0)),
        pl.BlockSpec((NUM_LANG, D), lambda i, b: (0, 0)),
        pl.BlockSpec((BLK, D), lambda i, b: (b * NBLK_SL + i, 0)),
        pl.BlockSpec((BLK, D), lambda i, b: (sl * NBLK_SL + i, 0)),
    ]
    out_spec = pl.BlockSpec((BLK, D),
                            lambda i, b: (b * NBLK + sl * NBLK_SL + i, 0))
    if aliased:
        specs = [pl.BlockSpec(
            (BLK, D), lambda i, b: (b * NBLK + sl * NBLK_SL + i, 0))] + specs
        kw["input_output_aliases"] = {0: 0}

        def body(prev_ref, lid_ref, ltab_ref, g_ref, pe_ref, o_ref):
            del prev_ref
            _combine_body(lid_ref, ltab_ref, g_ref, pe_ref, o_ref)
    else:
        body = _combine_body

    return pl.pallas_call(
        body,
        grid=(NBLK_SL, B),
        in_specs=specs,
        out_specs=out_spec,
        out_shape=jax.ShapeDtypeStruct((N, D), jnp.float32),
        **kw,
    )


_combine_slice = [_make_combine(sl, aliased=(sl > 0)) for sl in range(NSL)]


def kernel(token_ids, lang_ids, W_tok, W_lang, W_proj):
    lang_r = lang_ids.reshape(-1).astype(jnp.int32).reshape(B * NBLK, 1, BLK)
    ltab = _lang_proj(W_lang, W_proj)
    pe16 = jnp.asarray(_PE16)
    tok3 = token_ids.astype(jnp.int32).reshape(B, NSL, S_SL)
    g = [_sc_gather(W_tok, tok3[:, sl, :].reshape(-1)) for sl in range(NSL)]
    out = _combine_slice[0](lang_r, ltab, g[0], pe16)
    for sl in range(1, NSL):
        out = _combine_slice[sl](out, lang_r, ltab, g[sl], pe16)
    return out.reshape(B, S, D)


# speedup vs baseline: 1.0832x; 1.0832x over previous
import functools
import math

import jax
import jax.numpy as jnp
import ml_dtypes
import numpy as np
from jax import lax
from jax.experimental import pallas as pl
from jax.experimental.pallas import tpu as pltpu
from jax.experimental.pallas import tpu_sc as plsc

VOCAB = 100000
D = 768
NUM_LANG = 4
B = 4
S = 4096
N = B * S
SCALE = math.sqrt(D)

NSL = 2
S_SL = S // NSL
N_SL = B * S_SL

NC = 2
NS = 16
NW = NC * NS
R_PER_W = N_SL // NW
CHUNK = 32
NBUF = 4
PF = 2
NCHUNK = R_PER_W // CHUNK

BLK = 1024
NBLK_SL = S_SL // BLK
NBLK = S // BLK


def _pos_table():
    pos = np.arange(0, S, dtype=np.float32)[:, None]
    div = np.exp(np.arange(0, D, 2, dtype=np.float32) * (-math.log(10000.0) / D))
    pe = np.zeros((S, D), dtype=np.float32)
    pe[:, 0::2] = np.sin(pos * div)
    pe[:, 1::2] = np.cos(pos * div)
    return pe.astype(ml_dtypes.bfloat16)


_PE16 = _pos_table()


def _proj_body(wl_ref, wp_ref, o_ref):
    o_ref[...] = lax.dot_general(
        wl_ref[...], wp_ref[...], (((1,), (1,)), ((), ())),
        preferred_element_type=jnp.float32)


_lang_proj = pl.pallas_call(
    _proj_body,
    out_shape=jax.ShapeDtypeStruct((NUM_LANG, D), jnp.float32),
)


@functools.partial(
    pl.kernel,
    out_type=jax.ShapeDtypeStruct((N_SL, D), jnp.float32),
    mesh=plsc.VectorSubcoreMesh(core_axis_name="c", subcore_axis_name="s"),
    scratch_types=(
        [pltpu.VMEM((R_PER_W,), jnp.int32)]
        + [pltpu.VMEM((CHUNK, D), jnp.float32)] * NBUF
        + [pltpu.SemaphoreType.DMA] * (2 * NBUF)
    ),
)
def _sc_gather(wtok_hbm, tokid_hbm, g_hbm, idx_v, *bufs_and_sems):
    bufs = bufs_and_sems[:NBUF]
    gsems = bufs_and_sems[NBUF:2 * NBUF]
    ssems = bufs_and_sems[2 * NBUF:]
    wid = lax.axis_index("s") * NC + lax.axis_index("c")
    base = wid * R_PER_W
    pltpu.sync_copy(tokid_hbm.at[pl.ds(base, R_PER_W)], idx_v)
    g_cp = [None] * NCHUNK
    st_cp = [None] * NCHUNK
    for c in range(PF):
        g_cp[c] = pltpu.async_copy(
            wtok_hbm.at[idx_v.at[pl.ds(c * CHUNK, CHUNK)]], bufs[c % NBUF],
            gsems[c % NBUF])
    for c in range(NCHUNK):
        k = c % NBUF
        if c + PF < NCHUNK:
            if c >= NBUF - PF:
                st_cp[c - (NBUF - PF)].wait()
            kk = (c + PF) % NBUF
            g_cp[c + PF] = pltpu.async_copy(
                wtok_hbm.at[idx_v.at[pl.ds((c + PF) * CHUNK, CHUNK)]],
                bufs[kk], gsems[kk])
        g_cp[c].wait()
        st_cp[c] = pltpu.async_copy(
            bufs[k], g_hbm.at[pl.ds(base + c * CHUNK, CHUNK)], ssems[k])
    for c in range(max(0, NCHUNK - NBUF), NCHUNK):
        st_cp[c].wait()


def _combine_body(lid_ref, ltab_ref, g_ref, pe_ref, o_ref):
    ids_row = lid_ref[0]
    oh = (lax.broadcasted_iota(jnp.int32, (NUM_LANG, BLK), 0)
          == jnp.broadcast_to(ids_row, (NUM_LANG, BLK))).astype(jnp.float32)
    lang = lax.dot_general(oh, ltab_ref[...], (((0,), (0,)), ((), ())),
                           preferred_element_type=jnp.float32)
    o_ref[...] = (g_ref[...] * SCALE + lang
                  + pe_ref[...].astype(jnp.float32))


def _make_combine(sl, aliased):
    kw = {}
    specs = [
        pl.BlockSpec((1, 1, BLK),
                     lambda i, b: (b * NBLK + sl * NBLK_SL + i, 0, 0)),
        pl.BlockSpec((NUM_LANG, D), lambda i, b: (0, 0)),
        pl.BlockSpec((BLK, D), lambda i, b: (b * NBLK_SL + i, 0)),
        pl.BlockSpec((BLK, D), lambda i, b: (sl * NBLK_SL + i, 0)),
    ]
    out_spec = pl.BlockSpec((BLK, D),
                            lambda i, b: (b * NBLK + sl * NBLK_SL + i, 0))
    if aliased:
        specs = [pl.BlockSpec(
            (BLK, D), lambda i, b: (b * NBLK + sl * NBLK_SL + i, 0))] + specs
        kw["input_output_aliases"] = {0: 0}

        def body(prev_ref, lid_ref, ltab_ref, g_ref, pe_ref, o_ref):
            del prev_ref
            _combine_body(lid_ref, ltab_ref, g_ref, pe_ref, o_ref)
    else:
        body = _combine_body

    return pl.pallas_call(
        body,
        grid=(NBLK_SL, B),
        in_specs=specs,
        out_specs=out_spec,
        out_shape=jax.ShapeDtypeStruct((N, D), jnp.float32),
        **kw,
    )


_combine_slice = [_make_combine(sl, aliased=(sl > 0)) for sl in range(NSL)]


def kernel(token_ids, lang_ids, W_tok, W_lang, W_proj):
    lang_r = lang_ids.reshape(-1).astype(jnp.int32).reshape(B * NBLK, 1, BLK)
    ltab = _lang_proj(W_lang, W_proj)
    pe16 = jnp.asarray(_PE16)
    tok3 = token_ids.astype(jnp.int32).reshape(B, NSL, S_SL)
    g = [_sc_gather(W_tok, tok3[:, sl, :].reshape(-1)) for sl in range(NSL)]
    out = _combine_slice[0](lang_r, ltab, g[0], pe16)
    for sl in range(1, NSL):
        out = _combine_slice[sl](out, lang_r, ltab, g[sl], pe16)
    return out.reshape(B, S, D)


# speedup vs baseline: 1.0915x; 1.0076x over previous
import functools
import math

import jax
import jax.numpy as jnp
import ml_dtypes
import numpy as np
from jax import lax
from jax.experimental import pallas as pl
from jax.experimental.pallas import tpu as pltpu
from jax.experimental.pallas import tpu_sc as plsc

VOCAB = 100000
D = 768
NUM_LANG = 4
B = 4
S = 4096
N = B * S
SCALE = math.sqrt(D)

NSL = 2
S_SL = S // NSL
N_SL = B * S_SL

NC = 2
NS = 16
NW = NC * NS
R_PER_W = N_SL // NW
CHUNK = 32
NBUF = 4
PF = 2
NCHUNK = R_PER_W // CHUNK

BLK = 2048
NBLK_SL = S_SL // BLK
NBLK = S // BLK


def _pos_table():
    pos = np.arange(0, S, dtype=np.float32)[:, None]
    div = np.exp(np.arange(0, D, 2, dtype=np.float32) * (-math.log(10000.0) / D))
    pe = np.zeros((S, D), dtype=np.float32)
    pe[:, 0::2] = np.sin(pos * div)
    pe[:, 1::2] = np.cos(pos * div)
    return pe.astype(ml_dtypes.bfloat16)


_PE16 = _pos_table()


def _proj_body(wl_ref, wp_ref, o_ref):
    o_ref[...] = lax.dot_general(
        wl_ref[...], wp_ref[...], (((1,), (1,)), ((), ())),
        preferred_element_type=jnp.float32)


_lang_proj = pl.pallas_call(
    _proj_body,
    out_shape=jax.ShapeDtypeStruct((NUM_LANG, D), jnp.float32),
)


@functools.partial(
    pl.kernel,
    out_type=jax.ShapeDtypeStruct((N_SL, D), jnp.float32),
    mesh=plsc.VectorSubcoreMesh(core_axis_name="c", subcore_axis_name="s"),
    scratch_types=(
        [pltpu.VMEM((R_PER_W,), jnp.int32)]
        + [pltpu.VMEM((CHUNK, D), jnp.float32)] * NBUF
        + [pltpu.SemaphoreType.DMA] * (2 * NBUF)
    ),
)
def _sc_gather(wtok_hbm, tokid_hbm, g_hbm, idx_v, *bufs_and_sems):
    bufs = bufs_and_sems[:NBUF]
    gsems = bufs_and_sems[NBUF:2 * NBUF]
    ssems = bufs_and_sems[2 * NBUF:]
    wid = lax.axis_index("s") * NC + lax.axis_index("c")
    base = wid * R_PER_W
    pltpu.sync_copy(tokid_hbm.at[pl.ds(base, R_PER_W)], idx_v)
    g_cp = [None] * NCHUNK
    st_cp = [None] * NCHUNK
    for c in range(PF):
        g_cp[c] = pltpu.async_copy(
            wtok_hbm.at[idx_v.at[pl.ds(c * CHUNK, CHUNK)]], bufs[c % NBUF],
            gsems[c % NBUF])
    for c in range(NCHUNK):
        k = c % NBUF
        if c + PF < NCHUNK:
            if c >= NBUF - PF:
                st_cp[c - (NBUF - PF)].wait()
            kk = (c + PF) % NBUF
            g_cp[c + PF] = pltpu.async_copy(
                wtok_hbm.at[idx_v.at[pl.ds((c + PF) * CHUNK, CHUNK)]],
                bufs[kk], gsems[kk])
        g_cp[c].wait()
        st_cp[c] = pltpu.async_copy(
            bufs[k], g_hbm.at[pl.ds(base + c * CHUNK, CHUNK)], ssems[k])
    for c in range(max(0, NCHUNK - NBUF), NCHUNK):
        st_cp[c].wait()


def _combine_body(lid_ref, ltab_ref, g_ref, pe_ref, o_ref):
    ids_row = lid_ref[0]
    oh = (lax.broadcasted_iota(jnp.int32, (NUM_LANG, BLK), 0)
          == jnp.broadcast_to(ids_row, (NUM_LANG, BLK))).astype(jnp.float32)
    lang = lax.dot_general(oh, ltab_ref[...], (((0,), (0,)), ((), ())),
                           preferred_element_type=jnp.float32)
    o_ref[...] = (g_ref[...] * SCALE + lang
                  + pe_ref[...].astype(jnp.float32))


def _make_combine(sl, aliased):
    kw = {}
    specs = [
        pl.BlockSpec((1, 1, BLK),
                     lambda i, b: (b * NBLK + sl * NBLK_SL + i, 0, 0)),
        pl.BlockSpec((NUM_LANG, D), lambda i, b: (0, 0)),
        pl.BlockSpec((BLK, D), lambda i, b: (b * NBLK_SL + i, 0)),
        pl.BlockSpec((BLK, D), lambda i, b: (sl * NBLK_SL + i, 0)),
    ]
    out_spec = pl.BlockSpec((BLK, D),
                            lambda i, b: (b * NBLK + sl * NBLK_SL + i, 0))
    if aliased:
        specs = [pl.BlockSpec(
            (BLK, D), lambda i, b: (b * NBLK + sl * NBLK_SL + i, 0))] + specs
        kw["input_output_aliases"] = {0: 0}

        def body(prev_ref, lid_ref, ltab_ref, g_ref, pe_ref, o_ref):
            del prev_ref
            _combine_body(lid_ref, ltab_ref, g_ref, pe_ref, o_ref)
    else:
        body = _combine_body

    return pl.pallas_call(
        body,
        grid=(NBLK_SL, B),
        in_specs=specs,
        out_specs=out_spec,
        out_shape=jax.ShapeDtypeStruct((N, D), jnp.float32),
        **kw,
    )


_combine_slice = [_make_combine(sl, aliased=(sl > 0)) for sl in range(NSL)]


def kernel(token_ids, lang_ids, W_tok, W_lang, W_proj):
    lang_r = lang_ids.reshape(-1).astype(jnp.int32).reshape(B * NBLK, 1, BLK)
    ltab = _lang_proj(W_lang, W_proj)
    pe16 = jnp.asarray(_PE16)
    tok3 = token_ids.astype(jnp.int32).reshape(B, NSL, S_SL)
    g = [_sc_gather(W_tok, tok3[:, sl, :].reshape(-1)) for sl in range(NSL)]
    out = _combine_slice[0](lang_r, ltab, g[0], pe16)
    for sl in range(1, NSL):
        out = _combine_slice[sl](out, lang_r, ltab, g[sl], pe16)
    return out.reshape(B, S, D)
